# baseline (speedup 1.0000x reference)
import functools

import jax
import jax.numpy as jnp
from jax import lax
from jax.experimental import pallas as pl
from jax.experimental.pallas import tpu as pltpu
from jax.experimental.pallas import tpu_sc as plsc

N_NODES = 50000
N_EDGES = 800000

NC = 2
NS = 16
NW = NC * NS

NP = 50176
PSLICE = NP // NS
EPW = N_EDGES // NW
CHUNK = 5000
NCHUNK = EPW // CHUNK
ONES_PAD = 5008

_mesh = plsc.VectorSubcoreMesh(core_axis_name="c", subcore_axis_name="s")


def _zero_fill(buf, n):
    @pl.loop(0, n, step=16)
    def _(i):
        buf[pl.ds(i, 16)] = jnp.zeros((16,), jnp.float32)


def _deg_body(dst_hbm, degp_hbm, idxd, ones_v, zblk_v, acc_sh, sem_i, sem_s):
    c = lax.axis_index("c")
    s = lax.axis_index("s")
    base = (c * NS + s) * EPW

    cd = [pltpu.async_copy(dst_hbm.at[pl.ds(base + k * CHUNK, CHUNK)],
                           idxd[k], sem_i) for k in range(NCHUNK)]

    @pl.loop(0, ONES_PAD, step=16)
    def _(i):
        ones_v[pl.ds(i, 16)] = jnp.full((16,), 1.0, jnp.float32)

    _zero_fill(zblk_v, PSLICE)
    pltpu.sync_copy(zblk_v, acc_sh.at[pl.ds(s * PSLICE, PSLICE)])
    plsc.subcore_barrier()

    sc = []
    for k in range(NCHUNK):
        cd[k].wait()
        sc.append(pltpu.async_copy(ones_v.at[pl.ds(0, CHUNK)],
                                   acc_sh.at[idxd[k]], sem_s, add=True))
    for k in range(NCHUNK):
        sc[k].wait()

    plsc.subcore_barrier()
    sl = pl.ds(s * PSLICE, PSLICE)
    pltpu.sync_copy(acc_sh.at[sl], zblk_v)
    pltpu.sync_copy(zblk_v, degp_hbm.at[pl.ds(c * NP + s * PSLICE, PSLICE)])


def _pass_body(src_hbm, dst_hbm, val_hbm, outp_hbm,
               idxs, idxd, vals, zblk_v, acc_sh, val_sh,
               sem_i, sem_g, sem_s):
    c = lax.axis_index("c")
    s = lax.axis_index("s")
    sl = pl.ds(s * PSLICE, PSLICE)
    base = (c * NS + s) * EPW

    cs = [pltpu.async_copy(src_hbm.at[pl.ds(base + k * CHUNK, CHUNK)],
                           idxs[k], sem_i) for k in range(NCHUNK)]
    cd = [pltpu.async_copy(dst_hbm.at[pl.ds(base + k * CHUNK, CHUNK)],
                           idxd[k], sem_i) for k in range(NCHUNK)]

    _zero_fill(zblk_v, PSLICE)
    pltpu.sync_copy(zblk_v, acc_sh.at[sl])
    pltpu.sync_copy(val_hbm.at[sl], zblk_v)
    pltpu.sync_copy(zblk_v, val_sh.at[sl])
    plsc.subcore_barrier()

    gs = []
    for k in range(NCHUNK):
        cs[k].wait()
        gs.append(pltpu.async_copy(val_sh.at[idxs[k]], vals[k], sem_g))
    sc = []
    for k in range(NCHUNK):
        gs[k].wait()
        cd[k].wait()
        sc.append(pltpu.async_copy(vals[k], acc_sh.at[idxd[k]], sem_s, add=True))
    for k in range(NCHUNK):
        sc[k].wait()

    plsc.subcore_barrier()
    pltpu.sync_copy(acc_sh.at[sl], zblk_v)
    pltpu.sync_copy(zblk_v, outp_hbm.at[pl.ds(c * NP + s * PSLICE, PSLICE)])


_f32 = jnp.float32


@functools.partial(
    pl.kernel,
    out_type=jax.ShapeDtypeStruct((NC * NP,), _f32),
    mesh=_mesh,
    scratch_types=(
        [[pltpu.VMEM((CHUNK,), jnp.int32) for _ in range(NCHUNK)]]
        + [
            pltpu.VMEM((ONES_PAD,), _f32),
            pltpu.VMEM((PSLICE,), _f32),
            pltpu.VMEM_SHARED((NP,), _f32),
            pltpu.SemaphoreType.DMA,
            pltpu.SemaphoreType.DMA,
        ]
    ),
)
def _sc_degree(dst_hbm, degp_hbm, idxd, ones_v, zblk_v, acc_sh, sem_i, sem_s):
    _deg_body(dst_hbm, degp_hbm, idxd, ones_v, zblk_v, acc_sh, sem_i, sem_s)


@functools.partial(
    pl.kernel,
    out_type=jax.ShapeDtypeStruct((NC * NP,), _f32),
    mesh=_mesh,
    scratch_types=(
        [[pltpu.VMEM((CHUNK,), jnp.int32) for _ in range(NCHUNK)],
         [pltpu.VMEM((CHUNK,), jnp.int32) for _ in range(NCHUNK)],
         [pltpu.VMEM((CHUNK,), _f32) for _ in range(NCHUNK)]]
        + [
            pltpu.VMEM((PSLICE,), _f32),
            pltpu.VMEM_SHARED((NP,), _f32),
            pltpu.VMEM_SHARED((NP,), _f32),
            pltpu.SemaphoreType.DMA,
            pltpu.SemaphoreType.DMA,
            pltpu.SemaphoreType.DMA,
        ]
    ),
)
def _sc_pass(src_hbm, dst_hbm, val_hbm, outp_hbm,
             idxs, idxd, vals, zblk_v, acc_sh, val_sh, sem_i, sem_g, sem_s):
    _pass_body(src_hbm, dst_hbm, val_hbm, outp_hbm,
               idxs, idxd, vals, zblk_v, acc_sh, val_sh, sem_i, sem_g, sem_s)


def _tc_prep_body(d0_ref, d1_ref, x_ref, dinv_ref, u_ref):
    deg = d0_ref[...] + d1_ref[...] + 1.0
    dinv = lax.rsqrt(deg)
    dinv_ref[...] = dinv
    u_ref[...] = dinv * x_ref[...]


def _tc_mid_a_body(t0_ref, t1_ref, u_ref, dinv_ref, y_ref):
    y_ref[...] = dinv_ref[...] * (t0_ref[...] + t1_ref[...] + u_ref[...])


def _tc_mm_body(y8_ref, w1_ref, b1_ref, w2_ref, w8_ref):
    for j in range(8):
        h1 = y8_ref[:, j:j + 1] * w1_ref[...] + b1_ref[...]
        w8_ref[:, j:j + 1] = jnp.dot(h1, w2_ref[...],
                                     preferred_element_type=_f32)


def _tc_mid_b_body(dinv_ref, w_ref, v_ref):
    v_ref[...] = dinv_ref[...] * w_ref[...]


def _tc_fin_body(t0_ref, t1_ref, v_ref, dinv_ref, b2_ref, o_ref):
    o_ref[...] = (dinv_ref[...] * (t0_ref[...] + t1_ref[...] + v_ref[...])
                  + b2_ref[0])


_node1d = jax.ShapeDtypeStruct((NP,), _f32)
_MROWS = NP // 8

_tc_prep = pl.pallas_call(_tc_prep_body, out_shape=(_node1d, _node1d))
_tc_mid_a = pl.pallas_call(_tc_mid_a_body, out_shape=_node1d)
_tc_mm = pl.pallas_call(
    _tc_mm_body,
    out_shape=jax.ShapeDtypeStruct((_MROWS, 8), _f32),
)
_tc_mid_b = pl.pallas_call(_tc_mid_b_body, out_shape=_node1d)
_tc_fin = pl.pallas_call(_tc_fin_body, out_shape=_node1d)


def kernel(x, edge_index, W1, b1, W2, b2):
    ei = edge_index.astype(jnp.int32)
    dst = ei[1]
    src = lax.optimization_barrier(ei)[0]

    xp = jnp.zeros((NP,), _f32).at[:N_NODES].set(x[:, 0])
    w1 = W1.reshape(1, 64)
    b1r = b1.reshape(1, 64)

    degp = _sc_degree(dst)
    dinv, u = _tc_prep(degp[:NP], degp[NP:], xp)

    t1p = _sc_pass(src, dst, u)
    y = _tc_mid_a(t1p[:NP], t1p[NP:], u, dinv)

    w = _tc_mm(y.reshape(_MROWS, 8), w1, b1r, W2).reshape(NP)
    v = _tc_mid_b(dinv, w)

    t2p = _sc_pass(src, dst, v)
    out = _tc_fin(t2p[:NP], t2p[NP:], v, dinv, b2)

    return out[:N_NODES].reshape(N_NODES, 1)


# speedup vs baseline: 1.0063x; 1.0063x over previous
import functools

import jax
import jax.numpy as jnp
from jax import lax
from jax.experimental import pallas as pl
from jax.experimental.pallas import tpu as pltpu
from jax.experimental.pallas import tpu_sc as plsc

N_NODES = 50000
N_EDGES = 800000

NC = 2
NS = 16
NW = NC * NS

NP = 50176
PSLICE = NP // NS
EPW = N_EDGES // NW
CHUNK = 5000
NCHUNK = EPW // CHUNK
ONES_PAD = 5008

_mesh = plsc.VectorSubcoreMesh(core_axis_name="c", subcore_axis_name="s")


def _zero_fill(buf, n):
    @pl.loop(0, n, step=16)
    def _(i):
        buf[pl.ds(i, 16)] = jnp.zeros((16,), jnp.float32)


def _deg_body(dst_hbm, degp_hbm, idxd, ones_v, zblk_v, acc_sh, sem_i, sem_s):
    c = lax.axis_index("c")
    s = lax.axis_index("s")
    base = (c * NS + s) * EPW

    cd = [pltpu.async_copy(dst_hbm.at[pl.ds(base + k * CHUNK, CHUNK)],
                           idxd[k], sem_i) for k in range(NCHUNK)]

    @pl.loop(0, ONES_PAD, step=16)
    def _(i):
        ones_v[pl.ds(i, 16)] = jnp.full((16,), 1.0, jnp.float32)

    _zero_fill(zblk_v, PSLICE)
    pltpu.sync_copy(zblk_v, acc_sh.at[pl.ds(s * PSLICE, PSLICE)])
    plsc.subcore_barrier()

    sc = []
    for k in range(NCHUNK):
        cd[k].wait()
        sc.append(pltpu.async_copy(ones_v.at[pl.ds(0, CHUNK)],
                                   acc_sh.at[idxd[k]], sem_s, add=True))
    for k in range(NCHUNK):
        sc[k].wait()

    plsc.subcore_barrier()
    sl = pl.ds(s * PSLICE, PSLICE)
    pltpu.sync_copy(acc_sh.at[sl], zblk_v)
    pltpu.sync_copy(zblk_v, degp_hbm.at[pl.ds(c * NP + s * PSLICE, PSLICE)])


def _pass_body(src_hbm, dst_hbm, val_hbm, outp_hbm,
               idxs, idxd, vals, zblk_v, acc_sh, val_sh,
               sem_i, sem_g, sem_s):
    c = lax.axis_index("c")
    s = lax.axis_index("s")
    sl = pl.ds(s * PSLICE, PSLICE)
    base = (c * NS + s) * EPW

    cs = [pltpu.async_copy(src_hbm.at[pl.ds(base + k * CHUNK, CHUNK)],
                           idxs[k], sem_i) for k in range(NCHUNK)]
    cd = [pltpu.async_copy(dst_hbm.at[pl.ds(base + k * CHUNK, CHUNK)],
                           idxd[k], sem_i) for k in range(NCHUNK)]

    _zero_fill(zblk_v, PSLICE)
    pltpu.sync_copy(zblk_v, acc_sh.at[sl])
    pltpu.sync_copy(val_hbm.at[sl], zblk_v)
    pltpu.sync_copy(zblk_v, val_sh.at[sl])
    plsc.subcore_barrier()

    gs = []
    for k in range(NCHUNK):
        cs[k].wait()
        gs.append(pltpu.async_copy(val_sh.at[idxs[k]], vals[k], sem_g))
    sc = []
    for k in range(NCHUNK):
        gs[k].wait()
        cd[k].wait()
        sc.append(pltpu.async_copy(vals[k], acc_sh.at[idxd[k]], sem_s, add=True))
    for k in range(NCHUNK):
        sc[k].wait()

    plsc.subcore_barrier()
    pltpu.sync_copy(acc_sh.at[sl], zblk_v)
    pltpu.sync_copy(zblk_v, outp_hbm.at[pl.ds(c * NP + s * PSLICE, PSLICE)])


_f32 = jnp.float32


@functools.partial(
    pl.kernel,
    out_type=jax.ShapeDtypeStruct((NC * NP,), _f32),
    mesh=_mesh,
    scratch_types=(
        [[pltpu.VMEM((CHUNK,), jnp.int32) for _ in range(NCHUNK)]]
        + [
            pltpu.VMEM((ONES_PAD,), _f32),
            pltpu.VMEM((PSLICE,), _f32),
            pltpu.VMEM_SHARED((NP,), _f32),
            pltpu.SemaphoreType.DMA,
            pltpu.SemaphoreType.DMA,
        ]
    ),
)
def _sc_degree(dst_hbm, degp_hbm, idxd, ones_v, zblk_v, acc_sh, sem_i, sem_s):
    _deg_body(dst_hbm, degp_hbm, idxd, ones_v, zblk_v, acc_sh, sem_i, sem_s)


@functools.partial(
    pl.kernel,
    out_type=jax.ShapeDtypeStruct((NC * NP,), _f32),
    mesh=_mesh,
    scratch_types=(
        [[pltpu.VMEM((CHUNK,), jnp.int32) for _ in range(NCHUNK)],
         [pltpu.VMEM((CHUNK,), jnp.int32) for _ in range(NCHUNK)],
         [pltpu.VMEM((CHUNK,), _f32) for _ in range(NCHUNK)]]
        + [
            pltpu.VMEM((PSLICE,), _f32),
            pltpu.VMEM_SHARED((NP,), _f32),
            pltpu.VMEM_SHARED((NP,), _f32),
            pltpu.SemaphoreType.DMA,
            pltpu.SemaphoreType.DMA,
            pltpu.SemaphoreType.DMA,
        ]
    ),
)
def _sc_pass(src_hbm, dst_hbm, val_hbm, outp_hbm,
             idxs, idxd, vals, zblk_v, acc_sh, val_sh, sem_i, sem_g, sem_s):
    _pass_body(src_hbm, dst_hbm, val_hbm, outp_hbm,
               idxs, idxd, vals, zblk_v, acc_sh, val_sh, sem_i, sem_g, sem_s)


def _tc_prep_body(d0_ref, d1_ref, x_ref, dinv_ref, u_ref):
    deg = d0_ref[...] + d1_ref[...] + 1.0
    dinv = lax.rsqrt(deg)
    dinv_ref[...] = dinv
    u_ref[...] = dinv * x_ref[...]


def _tc_mid_a_body(t0_ref, t1_ref, u_ref, dinv_ref, y_ref):
    y_ref[...] = dinv_ref[...] * (t0_ref[...] + t1_ref[...] + u_ref[...])


def _tc_mm_body(y8_ref, w1_ref, b1_ref, w2_ref, w8_ref):
    for j in range(8):
        h1 = y8_ref[:, j:j + 1] * w1_ref[...] + b1_ref[...]
        w8_ref[:, j:j + 1] = jnp.dot(h1, w2_ref[...],
                                     preferred_element_type=_f32)


def _tc_mid_b_body(dinv_ref, w_ref, v_ref):
    v_ref[...] = dinv_ref[...] * w_ref[...]


def _tc_fin_body(t0_ref, t1_ref, v_ref, dinv_ref, b2_ref, o_ref):
    o_ref[...] = (dinv_ref[...] * (t0_ref[...] + t1_ref[...] + v_ref[...])
                  + b2_ref[0])


_node1d = jax.ShapeDtypeStruct((NP,), _f32)
_MROWS = NP // 8

_tc_prep = pl.pallas_call(_tc_prep_body, out_shape=(_node1d, _node1d))
_tc_mid_a = pl.pallas_call(_tc_mid_a_body, out_shape=_node1d)
_tc_mm = pl.pallas_call(
    _tc_mm_body,
    out_shape=jax.ShapeDtypeStruct((_MROWS, 8), _f32),
)
_tc_mid_b = pl.pallas_call(_tc_mid_b_body, out_shape=_node1d)
_tc_fin = pl.pallas_call(_tc_fin_body, out_shape=_node1d)


def kernel(x, edge_index, W1, b1, W2, b2):
    ei = edge_index.astype(jnp.int32)
    src = ei[0]
    dst = ei[1]

    xp = jnp.zeros((NP,), _f32).at[:N_NODES].set(x[:, 0])
    w1 = W1.reshape(1, 64)
    b1r = b1.reshape(1, 64)

    degp = _sc_degree(dst)
    dinv, u = _tc_prep(degp[:NP], degp[NP:], xp)

    t1p = _sc_pass(src, dst, u)
    y = _tc_mid_a(t1p[:NP], t1p[NP:], u, dinv)

    w = _tc_mm(y.reshape(_MROWS, 8), w1, b1r, W2).reshape(NP)
    v = _tc_mid_b(dinv, w)

    t2p = _sc_pass(src, dst, v)
    out = _tc_fin(t2p[:NP], t2p[NP:], v, dinv, b2)

    return out[:N_NODES].reshape(N_NODES, 1)
